# baseline (speedup 1.0000x reference)
import functools

import jax
import jax.numpy as jnp
from jax.experimental import pallas as pl
from jax.experimental.pallas import tpu as pltpu

_TN = 2048
_ROWS_PER_BLK = 8
_SEL_K = 50
_NS_ITERS = 24
_SENTINEL = 3.0e38


_CCH = 128
_SLOTS = 128


def _compact_candidates(g, s):
    r, npad = s.shape
    nch = npad // _CCH
    f32 = jnp.float32
    smin = jnp.min(s.reshape(r, nch, _CCH), axis=2)

    ib = jax.lax.bitcast_convert_type(smin, jnp.int32)
    mkey = jnp.where(ib >= 0, ib, jnp.int32(-2147483648) - ib)
    lo = jnp.min(mkey, axis=1, keepdims=True)
    hi = jnp.max(mkey, axis=1, keepdims=True)

    def mb(_, carry):
        lo, hi = carry
        mid = (lo >> 1) + (hi >> 1) + (lo & hi & 1)
        cnt = jnp.sum((mkey <= mid).astype(jnp.int32), axis=1, keepdims=True)
        pred = cnt >= _SEL_K
        return jnp.where(pred, lo, mid + 1), jnp.where(pred, mid, hi)

    tauk, _ = jax.lax.fori_loop(0, 32, mb, (lo, hi))
    is_cand = (mkey <= tauk).astype(f32)

    ci = jax.lax.broadcasted_iota(jnp.int32, (nch, nch), 0)
    cj = jax.lax.broadcasted_iota(jnp.int32, (nch, nch), 1)
    ut = jnp.clip((cj - ci).astype(f32), 0.0, 1.0)
    rank = jax.lax.dot_general(is_cand, ut, (((1,), (0,)), ((), ())),
                               preferred_element_type=f32)

    si = jax.lax.broadcasted_iota(jnp.int32, (r, _SLOTS, nch), 1).astype(f32)
    diff = rank[:, None, :] - si
    onehot = jnp.clip(1.0 - jnp.abs(diff), 0.0, 1.0)
    p = onehot * is_cand[:, None, :]

    s3 = s.reshape(r, nch, _CCH)
    g3 = g.reshape(r, nch, _CCH)
    dn = (((2,), (1,)), ((0,), (0,)))
    cs = jax.lax.dot_general(p, s3, dn, preferred_element_type=f32)
    cg = jax.lax.dot_general(p, g3, dn, preferred_element_type=f32)
    cs = cs.reshape(r, _SLOTS * _CCH)
    cg = cg.reshape(r, _SLOTS * _CCH)
    ncand = jnp.sum(is_cand, axis=1)[:, None, None]
    slot_of = jax.lax.broadcasted_iota(
        jnp.int32, (r, _SLOTS, _CCH), 1).astype(f32)
    deadf = jnp.clip(slot_of - ncand + 1.0, 0.0, 1.0)
    dead = deadf.reshape(r, _SLOTS * _CCH) > 0.5
    cs = jnp.where(dead, _SENTINEL, cs)
    return cg, cs


def _topk_rowsum(g, s):
    ibits = jax.lax.bitcast_convert_type(s, jnp.int32)
    key = jnp.where(ibits >= 0, ibits, jnp.int32(-2147483648) - ibits)
    lo = jnp.min(key, axis=1, keepdims=True)
    hi = jnp.max(key, axis=1, keepdims=True)

    def bs_body(_, carry):
        lo, hi = carry
        mid = (lo >> 1) + (hi >> 1) + (lo & hi & 1)
        cnt = jnp.sum((key <= mid).astype(jnp.int32), axis=1, keepdims=True)
        pred = cnt >= _SEL_K
        return jnp.where(pred, lo, mid + 1), jnp.where(pred, mid, hi)

    t, _ = jax.lax.fori_loop(0, 32, bs_body, (lo, hi))
    lt = key < t
    eq = key == t
    cnt_lt = jnp.sum(lt.astype(jnp.float32), axis=1, keepdims=True)
    sum_lt = jnp.sum(jnp.where(lt, g, 0.0), axis=1, keepdims=True)
    cnt_eq = jnp.sum(eq.astype(jnp.float32), axis=1, keepdims=True)
    sum_eq = jnp.sum(jnp.where(eq, g, 0.0), axis=1, keepdims=True)
    rowsum = sum_lt + (_SEL_K - cnt_lt) * sum_eq / cnt_eq
    return jnp.sum(rowsum)


def _fused_body(c_ref, e_ref, x_ref, w_ref,
                orig_ref, y_ref, cp_ref, l2_ref, nm_ref, l1_ref,
                g_acc, esq_acc, *, n_valid, tn, n_tiles, n_concepts):
    j = pl.program_id(0)
    ct = c_ref[...]
    e = e_ref[...]
    f32 = jnp.float32
    g = jax.lax.dot_general(ct, e, (((1,), (0,)), ((), ())),
                            preferred_element_type=f32)
    esq = jnp.sum(e * e, axis=0, keepdims=True)
    col = jax.lax.broadcasted_iota(jnp.int32, g.shape, 1) + j * tn
    valid = col < n_valid
    g_acc[:, pl.ds(j * tn, tn)] = jnp.where(valid, g, 0.0)
    col1 = jax.lax.broadcasted_iota(jnp.int32, esq.shape, 1) + j * tn
    esq_acc[:, pl.ds(j * tn, tn)] = jnp.where(col1 < n_valid, esq, _SENTINEL)

    @pl.when(j == 0)
    def _dense():
        x = x_ref[...]
        w = w_ref[...]
        a = jax.lax.dot_general(ct, ct, (((1,), (1,)), ((), ())),
                                preferred_element_type=f32)
        r1 = jnp.max(jnp.sum(jnp.abs(a), axis=1))
        xinv = a * (1.0 / (r1 * r1))
        ii = jax.lax.broadcasted_iota(jnp.int32, a.shape, 0)
        jj = jax.lax.broadcasted_iota(jnp.int32, a.shape, 1)
        eye = (ii == jj).astype(f32)
        for _ in range(_NS_ITERS):
            axk = jax.lax.dot_general(a, xinv, (((1,), (0,)), ((), ())),
                                      preferred_element_type=f32)
            xinv = jax.lax.dot_general(xinv, 2.0 * eye - axk,
                                       (((1,), (0,)), ((), ())),
                                       preferred_element_type=f32)
        m1 = jax.lax.dot_general(x, ct, (((1,), (1,)), ((), ())),
                                 preferred_element_type=f32)
        m2 = jax.lax.dot_general(ct, w, (((1,), (0,)), ((), ())),
                                 preferred_element_type=f32)
        m1x = jax.lax.dot_general(m1, xinv, (((1,), (0,)), ((), ())),
                                  preferred_element_type=f32)
        y_ref[...] = jax.lax.dot_general(m1x, m2, (((1,), (0,)), ((), ())),
                                         preferred_element_type=f32)
        orig_ref[...] = jax.lax.dot_general(x, w, (((1,), (0,)), ((), ())),
                                            preferred_element_type=f32)
        cp_ref[...] = jax.lax.dot_general(ct, x, (((0,), (0,)), ((), ())),
                                          preferred_element_type=f32)
        tr = jnp.sum(a * eye)
        tot = jnp.sum(a)
        denom = f32(n_concepts * n_concepts)
        l2_ref[...] = jnp.full((1, 1), (tot - tr) / denom, dtype=f32)
        nm_ref[...] = jnp.full((1, 1), tr / denom, dtype=f32)

    @pl.when(j == n_tiles - 1)
    def _select():
        esq_row = esq_acc[...]
        acc = jnp.float32(0.0)
        for blk in range(n_concepts // _ROWS_PER_BLK):
            gb = g_acc[pl.ds(blk * _ROWS_PER_BLK, _ROWS_PER_BLK), :]
            sb = esq_row - 2.0 * gb
            if sb.shape[1] // _CCH >= 4 * _SLOTS:
                gb, sb = _compact_candidates(gb, sb)
            acc = acc + _topk_rowsum(gb, sb)
        l1_ref[...] = jnp.full((1, 1), acc, dtype=jnp.float32)


def kernel(train_embedding, concept, train_embeddings_T, W_head, topk):
    bs, d = train_embedding.shape
    nc = concept.shape[1]
    n = train_embeddings_T.shape[1]
    ncls = W_head.shape[1]
    n_tiles = (n + _TN - 1) // _TN
    np_ = n_tiles * _TN

    (orig_pred, y_pred, concept_pred, l2, nm, l1_raw) = pl.pallas_call(
        functools.partial(_fused_body, n_valid=n, tn=_TN, n_tiles=n_tiles,
                          n_concepts=nc),
        grid=(n_tiles,),
        in_specs=[
            pl.BlockSpec((nc, d), lambda j: (0, 0)),
            pl.BlockSpec((d, _TN), lambda j: (0, j)),
            pl.BlockSpec((bs, d), lambda j: (0, 0)),
            pl.BlockSpec((d, ncls), lambda j: (0, 0)),
        ],
        out_specs=[
            pl.BlockSpec((bs, ncls), lambda j: (0, 0)),
            pl.BlockSpec((bs, ncls), lambda j: (0, 0)),
            pl.BlockSpec((d, d), lambda j: (0, 0)),
            pl.BlockSpec((1, 1), lambda j: (0, 0)),
            pl.BlockSpec((1, 1), lambda j: (0, 0)),
            pl.BlockSpec((1, 1), lambda j: (0, 0)),
        ],
        out_shape=[
            jax.ShapeDtypeStruct((bs, ncls), jnp.float32),
            jax.ShapeDtypeStruct((bs, ncls), jnp.float32),
            jax.ShapeDtypeStruct((d, d), jnp.float32),
            jax.ShapeDtypeStruct((1, 1), jnp.float32),
            jax.ShapeDtypeStruct((1, 1), jnp.float32),
            jax.ShapeDtypeStruct((1, 1), jnp.float32),
        ],
        scratch_shapes=[
            pltpu.VMEM((nc, np_), jnp.float32),
            pltpu.VMEM((1, np_), jnp.float32),
        ],
    )(concept.T, train_embeddings_T, train_embedding, W_head)

    l_sparse_1 = l1_raw[0, 0] * (jnp.float32(1.0) / (topk * nc))
    return (orig_pred, y_pred, l_sparse_1,
            l2[0, 0], nm[0, 0], concept_pred)


# speedup vs baseline: 1.2645x; 1.2645x over previous
import functools

import jax
import jax.numpy as jnp
from jax.experimental import pallas as pl
from jax.experimental.pallas import tpu as pltpu

_TN = 2048
_ROWS_PER_BLK = 8
_SEL_K = 50
_NS_ITERS = 24
_SENTINEL = 3.0e38


def _topk_rowsum(g, s):
    ibits = jax.lax.bitcast_convert_type(s, jnp.int32)
    key = jnp.where(ibits >= 0, ibits, jnp.int32(-2147483648) - ibits)
    lo = jnp.min(key, axis=1, keepdims=True)
    hi = jnp.max(key, axis=1, keepdims=True)

    def bs_body(_, carry):
        lo, hi = carry
        mid = (lo >> 1) + (hi >> 1) + (lo & hi & 1)
        cnt = jnp.sum((key <= mid).astype(jnp.int32), axis=1, keepdims=True)
        pred = cnt >= _SEL_K
        return jnp.where(pred, lo, mid + 1), jnp.where(pred, mid, hi)

    t, _ = jax.lax.fori_loop(0, 32, bs_body, (lo, hi))
    lt = key < t
    eq = key == t
    cnt_lt = jnp.sum(lt.astype(jnp.float32), axis=1, keepdims=True)
    sum_lt = jnp.sum(jnp.where(lt, g, 0.0), axis=1, keepdims=True)
    cnt_eq = jnp.sum(eq.astype(jnp.float32), axis=1, keepdims=True)
    sum_eq = jnp.sum(jnp.where(eq, g, 0.0), axis=1, keepdims=True)
    rowsum = sum_lt + (_SEL_K - cnt_lt) * sum_eq / cnt_eq
    return jnp.sum(rowsum)


def _fused_body(c_ref, e_ref, x_ref, w_ref,
                orig_ref, y_ref, cp_ref, l2_ref, nm_ref, l1_ref,
                g_acc, esq_acc, *, n_valid, tn, n_tiles, n_concepts):
    j = pl.program_id(0)
    ct = c_ref[...]
    e = e_ref[...]
    f32 = jnp.float32
    g = jax.lax.dot_general(ct, e, (((1,), (0,)), ((), ())),
                            preferred_element_type=f32)
    esq = jnp.sum(e * e, axis=0, keepdims=True)
    col = jax.lax.broadcasted_iota(jnp.int32, g.shape, 1) + j * tn
    valid = col < n_valid
    g_acc[:, pl.ds(j * tn, tn)] = jnp.where(valid, g, 0.0)
    col1 = jax.lax.broadcasted_iota(jnp.int32, esq.shape, 1) + j * tn
    esq_acc[:, pl.ds(j * tn, tn)] = jnp.where(col1 < n_valid, esq, _SENTINEL)

    @pl.when(j == 0)
    def _dense():
        x = x_ref[...]
        w = w_ref[...]
        a = jax.lax.dot_general(ct, ct, (((1,), (1,)), ((), ())),
                                preferred_element_type=f32)
        r1 = jnp.max(jnp.sum(jnp.abs(a), axis=1))
        xinv = a * (1.0 / (r1 * r1))
        ii = jax.lax.broadcasted_iota(jnp.int32, a.shape, 0)
        jj = jax.lax.broadcasted_iota(jnp.int32, a.shape, 1)
        eye = (ii == jj).astype(f32)
        for _ in range(_NS_ITERS):
            axk = jax.lax.dot_general(a, xinv, (((1,), (0,)), ((), ())),
                                      preferred_element_type=f32)
            xinv = jax.lax.dot_general(xinv, 2.0 * eye - axk,
                                       (((1,), (0,)), ((), ())),
                                       preferred_element_type=f32)
        m1 = jax.lax.dot_general(x, ct, (((1,), (1,)), ((), ())),
                                 preferred_element_type=f32)
        m2 = jax.lax.dot_general(ct, w, (((1,), (0,)), ((), ())),
                                 preferred_element_type=f32)
        m1x = jax.lax.dot_general(m1, xinv, (((1,), (0,)), ((), ())),
                                  preferred_element_type=f32)
        y_ref[...] = jax.lax.dot_general(m1x, m2, (((1,), (0,)), ((), ())),
                                         preferred_element_type=f32)
        orig_ref[...] = jax.lax.dot_general(x, w, (((1,), (0,)), ((), ())),
                                            preferred_element_type=f32)
        cp_ref[...] = jax.lax.dot_general(ct, x, (((0,), (0,)), ((), ())),
                                          preferred_element_type=f32)
        tr = jnp.sum(a * eye)
        tot = jnp.sum(a)
        denom = f32(n_concepts * n_concepts)
        l2_ref[...] = jnp.full((1, 1), (tot - tr) / denom, dtype=f32)
        nm_ref[...] = jnp.full((1, 1), tr / denom, dtype=f32)

    @pl.when(j == n_tiles - 1)
    def _select():
        esq_row = esq_acc[...]
        acc = jnp.float32(0.0)
        for blk in range(n_concepts // _ROWS_PER_BLK):
            gb = g_acc[pl.ds(blk * _ROWS_PER_BLK, _ROWS_PER_BLK), :]
            sb = esq_row - 2.0 * gb
            acc = acc + _topk_rowsum(gb, sb)
        l1_ref[...] = jnp.full((1, 1), acc, dtype=jnp.float32)


def kernel(train_embedding, concept, train_embeddings_T, W_head, topk):
    bs, d = train_embedding.shape
    nc = concept.shape[1]
    n = train_embeddings_T.shape[1]
    ncls = W_head.shape[1]
    n_tiles = (n + _TN - 1) // _TN
    np_ = n_tiles * _TN

    (orig_pred, y_pred, concept_pred, l2, nm, l1_raw) = pl.pallas_call(
        functools.partial(_fused_body, n_valid=n, tn=_TN, n_tiles=n_tiles,
                          n_concepts=nc),
        grid=(n_tiles,),
        in_specs=[
            pl.BlockSpec((nc, d), lambda j: (0, 0)),
            pl.BlockSpec((d, _TN), lambda j: (0, j)),
            pl.BlockSpec((bs, d), lambda j: (0, 0)),
            pl.BlockSpec((d, ncls), lambda j: (0, 0)),
        ],
        out_specs=[
            pl.BlockSpec((bs, ncls), lambda j: (0, 0)),
            pl.BlockSpec((bs, ncls), lambda j: (0, 0)),
            pl.BlockSpec((d, d), lambda j: (0, 0)),
            pl.BlockSpec((1, 1), lambda j: (0, 0)),
            pl.BlockSpec((1, 1), lambda j: (0, 0)),
            pl.BlockSpec((1, 1), lambda j: (0, 0)),
        ],
        out_shape=[
            jax.ShapeDtypeStruct((bs, ncls), jnp.float32),
            jax.ShapeDtypeStruct((bs, ncls), jnp.float32),
            jax.ShapeDtypeStruct((d, d), jnp.float32),
            jax.ShapeDtypeStruct((1, 1), jnp.float32),
            jax.ShapeDtypeStruct((1, 1), jnp.float32),
            jax.ShapeDtypeStruct((1, 1), jnp.float32),
        ],
        scratch_shapes=[
            pltpu.VMEM((nc, np_), jnp.float32),
            pltpu.VMEM((1, np_), jnp.float32),
        ],
    )(concept.T, train_embeddings_T, train_embedding, W_head)

    l_sparse_1 = l1_raw[0, 0] * (jnp.float32(1.0) / (topk * nc))
    return (orig_pred, y_pred, l_sparse_1,
            l2[0, 0], nm[0, 0], concept_pred)


# speedup vs baseline: 1.6188x; 1.2802x over previous
import functools

import jax
import jax.numpy as jnp
from jax.experimental import pallas as pl
from jax.experimental.pallas import tpu as pltpu

_TN = 2048
_ROWS_PER_BLK = 16
_SEL_K = 50
_NS_ITERS = 24
_SENTINEL = 3.0e38


def _topk_rowsum(g, s):
    ibits = jax.lax.bitcast_convert_type(s, jnp.int32)
    key = jnp.where(ibits >= 0, ibits, jnp.int32(-2147483648) - ibits)
    lo = jnp.min(key, axis=1, keepdims=True)
    hi = jnp.max(key, axis=1, keepdims=True)

    def bs_body(_, carry):
        lo, hi = carry
        mid = (lo >> 1) + (hi >> 1) + (lo & hi & 1)
        cnt = jnp.sum((key <= mid).astype(jnp.int32), axis=1, keepdims=True)
        pred = cnt >= _SEL_K
        return jnp.where(pred, lo, mid + 1), jnp.where(pred, mid, hi)

    t, _ = jax.lax.fori_loop(0, 32, bs_body, (lo, hi))
    lt = key < t
    eq = key == t
    cnt_lt = jnp.sum(lt.astype(jnp.float32), axis=1, keepdims=True)
    sum_lt = jnp.sum(jnp.where(lt, g, 0.0), axis=1, keepdims=True)
    cnt_eq = jnp.sum(eq.astype(jnp.float32), axis=1, keepdims=True)
    sum_eq = jnp.sum(jnp.where(eq, g, 0.0), axis=1, keepdims=True)
    rowsum = sum_lt + (_SEL_K - cnt_lt) * sum_eq / cnt_eq
    return jnp.sum(rowsum)


def _fused_body(c_ref, e_ref, x_ref, w_ref,
                orig_ref, y_ref, cp_ref, l2_ref, nm_ref, l1_ref,
                g_acc, esq_acc, *, n_valid, tn, n_tiles, n_concepts):
    j = pl.program_id(0)
    ct = c_ref[...]
    e = e_ref[...]
    f32 = jnp.float32
    g = jax.lax.dot_general(ct, e, (((1,), (0,)), ((), ())),
                            preferred_element_type=f32)
    esq = jnp.sum(e * e, axis=0, keepdims=True)
    col = jax.lax.broadcasted_iota(jnp.int32, g.shape, 1) + j * tn
    valid = col < n_valid
    g_acc[:, pl.ds(j * tn, tn)] = jnp.where(valid, g, 0.0)
    col1 = jax.lax.broadcasted_iota(jnp.int32, esq.shape, 1) + j * tn
    esq_acc[:, pl.ds(j * tn, tn)] = jnp.where(col1 < n_valid, esq, _SENTINEL)

    @pl.when(j == 0)
    def _dense():
        x = x_ref[...]
        w = w_ref[...]
        a = jax.lax.dot_general(ct, ct, (((1,), (1,)), ((), ())),
                                preferred_element_type=f32)
        r1 = jnp.max(jnp.sum(jnp.abs(a), axis=1))
        xinv = a * (1.0 / (r1 * r1))
        ii = jax.lax.broadcasted_iota(jnp.int32, a.shape, 0)
        jj = jax.lax.broadcasted_iota(jnp.int32, a.shape, 1)
        eye = (ii == jj).astype(f32)
        for _ in range(_NS_ITERS):
            axk = jax.lax.dot_general(a, xinv, (((1,), (0,)), ((), ())),
                                      preferred_element_type=f32)
            xinv = jax.lax.dot_general(xinv, 2.0 * eye - axk,
                                       (((1,), (0,)), ((), ())),
                                       preferred_element_type=f32)
        m1 = jax.lax.dot_general(x, ct, (((1,), (1,)), ((), ())),
                                 preferred_element_type=f32)
        m2 = jax.lax.dot_general(ct, w, (((1,), (0,)), ((), ())),
                                 preferred_element_type=f32)
        m1x = jax.lax.dot_general(m1, xinv, (((1,), (0,)), ((), ())),
                                  preferred_element_type=f32)
        y_ref[...] = jax.lax.dot_general(m1x, m2, (((1,), (0,)), ((), ())),
                                         preferred_element_type=f32)
        orig_ref[...] = jax.lax.dot_general(x, w, (((1,), (0,)), ((), ())),
                                            preferred_element_type=f32)
        cp_ref[...] = jax.lax.dot_general(ct, x, (((0,), (0,)), ((), ())),
                                          preferred_element_type=f32)
        tr = jnp.sum(a * eye)
        tot = jnp.sum(a)
        denom = f32(n_concepts * n_concepts)
        l2_ref[...] = jnp.full((1, 1), (tot - tr) / denom, dtype=f32)
        nm_ref[...] = jnp.full((1, 1), tr / denom, dtype=f32)

    @pl.when(j == n_tiles - 1)
    def _select():
        esq_row = esq_acc[...]
        acc = jnp.float32(0.0)
        for blk in range(n_concepts // _ROWS_PER_BLK):
            gb = g_acc[pl.ds(blk * _ROWS_PER_BLK, _ROWS_PER_BLK), :]
            sb = esq_row - 2.0 * gb
            acc = acc + _topk_rowsum(gb, sb)
        l1_ref[...] = jnp.full((1, 1), acc, dtype=jnp.float32)


def kernel(train_embedding, concept, train_embeddings_T, W_head, topk):
    bs, d = train_embedding.shape
    nc = concept.shape[1]
    n = train_embeddings_T.shape[1]
    ncls = W_head.shape[1]
    n_tiles = (n + _TN - 1) // _TN
    np_ = n_tiles * _TN

    (orig_pred, y_pred, concept_pred, l2, nm, l1_raw) = pl.pallas_call(
        functools.partial(_fused_body, n_valid=n, tn=_TN, n_tiles=n_tiles,
                          n_concepts=nc),
        grid=(n_tiles,),
        in_specs=[
            pl.BlockSpec((nc, d), lambda j: (0, 0)),
            pl.BlockSpec((d, _TN), lambda j: (0, j)),
            pl.BlockSpec((bs, d), lambda j: (0, 0)),
            pl.BlockSpec((d, ncls), lambda j: (0, 0)),
        ],
        out_specs=[
            pl.BlockSpec((bs, ncls), lambda j: (0, 0)),
            pl.BlockSpec((bs, ncls), lambda j: (0, 0)),
            pl.BlockSpec((d, d), lambda j: (0, 0)),
            pl.BlockSpec((1, 1), lambda j: (0, 0)),
            pl.BlockSpec((1, 1), lambda j: (0, 0)),
            pl.BlockSpec((1, 1), lambda j: (0, 0)),
        ],
        out_shape=[
            jax.ShapeDtypeStruct((bs, ncls), jnp.float32),
            jax.ShapeDtypeStruct((bs, ncls), jnp.float32),
            jax.ShapeDtypeStruct((d, d), jnp.float32),
            jax.ShapeDtypeStruct((1, 1), jnp.float32),
            jax.ShapeDtypeStruct((1, 1), jnp.float32),
            jax.ShapeDtypeStruct((1, 1), jnp.float32),
        ],
        scratch_shapes=[
            pltpu.VMEM((nc, np_), jnp.float32),
            pltpu.VMEM((1, np_), jnp.float32),
        ],
    )(concept.T, train_embeddings_T, train_embedding, W_head)

    l_sparse_1 = l1_raw[0, 0] * (jnp.float32(1.0) / (topk * nc))
    return (orig_pred, y_pred, l_sparse_1,
            l2[0, 0], nm[0, 0], concept_pred)
